# baseline (speedup 1.0000x reference)
import jax
import jax.numpy as jnp
from jax import lax
from jax.experimental import pallas as pl
from jax.experimental.pallas import tpu as pltpu
from jax.experimental.pallas import tpu_sc as plsc

N_NODES = 100000
D = 128
DH = 64
N1 = 50000
M1 = 20000
E1 = 320000
M2 = 10000
E2 = 160000

NS = 16
NC = 2
NW = NC * NS
CH = 128
NB = 4
SB = 16


def _cdiv(a, b):
    return (a + b - 1) // b


def _wait(src, dst, sem):
    pltpu.make_async_copy(src, dst, sem).wait()


def _make_row_gather(n_rows_out):
    kt = n_rows_out // NW
    GC = kt // 4
    assert kt % 4 == 0 and GC % 8 == 0
    mesh = plsc.VectorSubcoreMesh(core_axis_name="c", subcore_axis_name="s")

    def body(table, idx2, out, idx_v, r0, r1, s0, s1, g0, g1):
        rows = [r0, r1]
        wsem = [s0, s1]
        gsem = [g0, g1]
        wid = lax.axis_index("c") * NS + lax.axis_index("s")
        obase = wid * kt
        pltpu.sync_copy(idx2.at[wid], idx_v)

        def owrite(q):
            u = q % 2
            _wait(table.at[idx_v.at[pl.ds(0, GC)]], rows[u], gsem[u])
            pltpu.async_copy(rows[u], out.at[pl.ds(obase + q * GC, GC)],
                             wsem[u])

        for q in range(4):
            u = q % 2
            if q >= 2:
                _wait(rows[u], out.at[pl.ds(obase, GC)], wsem[u])
            pltpu.async_copy(table.at[idx_v.at[pl.ds(q * GC, GC)]],
                             rows[u], gsem[u])
            if q >= 1:
                owrite(q - 1)
        owrite(3)
        for q in range(2, 4):
            _wait(rows[q % 2], out.at[pl.ds(obase, GC)], wsem[q % 2])

    return pl.kernel(
        body,
        out_type=jax.ShapeDtypeStruct((n_rows_out, D), jnp.float32),
        mesh=mesh,
        compiler_params=pltpu.CompilerParams(use_tc_tiling_on_sc=False),
        scratch_types=(
            [pltpu.VMEM((kt,), jnp.int32)]
            + [pltpu.VMEM((GC, D), jnp.float32)] * 2
            + [pltpu.SemaphoreType.DMA] * 4
        ),
    )


def _make_edge_kernel(n_edges, n_dst_pad, n_res_pad):
    ke = n_edges // (NS * CH)
    nb = ke // SB
    assert ke % SB == 0 and SB % NB == 0
    kr = n_res_pad // (NS * CH)
    stripe = n_dst_pad // NS
    ks = stripe // CH
    assert kr == ks and kr >= NB and kr <= SB

    mesh = plsc.VectorSubcoreMesh(core_axis_name="c", subcore_axis_name="s")

    out_type = (
        jax.ShapeDtypeStruct((NC, n_dst_pad, DH), jnp.float32),
        jax.ShapeDtypeStruct((n_dst_pad, 8), jnp.float32),
        jax.ShapeDtypeStruct((NC, n_res_pad, DH), jnp.float32),
    )

    def body(table, src3, dst3, res3, z64, o8,
             agg_out, deg_out, xdst_out,
             src_b, dst_b, r0, r1, r2, r3, ones_v,
             agg_sh, deg_sh, s0, s1, s2, s3, g0, g1, g2, g3):
        rows = [r0, r1, r2, r3]
        ssem = [s0, s1, s2, s3]
        gsem = [g0, g1, g2, g3]
        cid = lax.axis_index("c")
        sid = lax.axis_index("s")
        base = sid * stripe

        pltpu.sync_copy(z64, rows[0])
        for r in range(ks):
            pltpu.sync_copy(rows[0], agg_sh.at[pl.ds(base + r * CH, CH)])

        @pl.when(cid == 0)
        def _():
            for r in range(ks):
                pltpu.sync_copy(o8.at[1], deg_sh.at[pl.ds(base + r * CH, CH)])
        pltpu.sync_copy(o8.at[0], ones_v)

        def compose(idx_v, nk):
            def jbody(j, _):
                for i in range(CH // 16):
                    s16 = idx_v[j, pl.ds(i * 16, 16)]
                    idx_v[j, pl.ds(i * 16, 16)] = s16 * 2 + cid
                return 0
            lax.fori_loop(0, nk, jbody, 0)

        plsc.subcore_barrier()

        def consume(jj, u):
            _wait(table.at[src_b.at[0]], rows[u], gsem[u])
            pltpu.async_copy(rows[u], agg_sh.at[dst_b.at[jj]], ssem[u],
                             add=True)

            @pl.when(cid == 0)
            def _():
                pltpu.sync_copy(ones_v, deg_sh.at[dst_b.at[jj]], add=True)

        def block_body(b, _):
            @pl.when(b >= 1)
            def _():
                consume(SB - 1, (SB - 1) % NB)

            pltpu.sync_copy(src3.at[sid, pl.ds(b * SB, SB)], src_b)
            pltpu.sync_copy(dst3.at[sid, pl.ds(b * SB, SB)], dst_b)
            compose(src_b, SB)

            def group_body(g, _):
                for u in range(NB):
                    j2 = g * NB + u
                    j = b * SB + j2

                    @pl.when(j >= NB)
                    def _():
                        _wait(rows[u], agg_sh.at[dst_b.at[0]], ssem[u])

                    pltpu.async_copy(table.at[src_b.at[j2]], rows[u], gsem[u])
                    if u > 0:
                        consume(j2 - 1, (u + NB - 1) % NB)
                    else:
                        @pl.when(g >= 1)
                        def _():
                            consume(j2 - 1, NB - 1)
                return 0
            lax.fori_loop(0, SB // NB, group_body, 0)
            return 0
        lax.fori_loop(0, nb, block_body, 0)

        consume(SB - 1, (SB - 1) % NB)
        for u in range(NB):
            _wait(rows[u], agg_sh.at[dst_b.at[0]], ssem[u])

        pltpu.sync_copy(res3.at[sid], src_b.at[pl.ds(0, kr)])
        compose(src_b, kr)

        def rwrite(j):
            u = j % NB
            _wait(table.at[src_b.at[0]], rows[u], gsem[u])
            pltpu.async_copy(rows[u],
                             xdst_out.at[cid, pl.ds(base + j * CH, CH)],
                             ssem[u])

        for j in range(kr):
            u = j % NB
            if j >= NB:
                _wait(rows[u], xdst_out.at[cid, pl.ds(base, CH)], ssem[u])
            pltpu.async_copy(table.at[src_b.at[j]], rows[u], gsem[u])
            if j >= 1:
                rwrite(j - 1)
        rwrite(kr - 1)
        for j in range(kr - NB, kr):
            u = j % NB
            _wait(rows[u], xdst_out.at[cid, pl.ds(base, CH)], ssem[u])

        plsc.subcore_barrier()

        pltpu.sync_copy(agg_sh.at[pl.ds(base, stripe)],
                        agg_out.at[cid, pl.ds(base, stripe)])

        @pl.when(cid == 0)
        def _():
            pltpu.sync_copy(deg_sh.at[pl.ds(base, stripe)],
                            deg_out.at[pl.ds(base, stripe)])

    return pl.kernel(
        body,
        out_type=out_type,
        mesh=mesh,
        compiler_params=pltpu.CompilerParams(use_tc_tiling_on_sc=False),
        scratch_types=(
            [pltpu.VMEM((SB, CH), jnp.int32),
             pltpu.VMEM((SB, CH), jnp.int32)]
            + [pltpu.VMEM((CH, DH), jnp.float32)] * NB
            + [pltpu.VMEM((CH, 8), jnp.float32),
               pltpu.VMEM_SHARED((n_dst_pad, DH), jnp.float32),
               pltpu.VMEM_SHARED((n_dst_pad, 8), jnp.float32)]
            + [pltpu.SemaphoreType.DMA] * (2 * NB)
        ),
    )


def _pad_reshape_idx(idx, total, fill, nmaj=NS):
    pad = total - idx.shape[0]
    idx = jnp.concatenate([idx.astype(jnp.int32),
                           jnp.full((pad,), fill, jnp.int32)])
    return idx.reshape(nmaj, total // (nmaj * CH), CH)


def _make_dense_kernel(n_rows, final_softmax):
    blk = 256
    grid = (n_rows // blk,)

    def body(alo, ahi, deg, xlo, xhi, w, wr, b, out):
        scale = 1.0 / jnp.maximum(deg[:, 0:1], 1.0)
        h = (jnp.dot(alo[0] * scale, w[0:DH, :], preferred_element_type=jnp.float32)
             + jnp.dot(ahi[0] * scale, w[DH:D, :], preferred_element_type=jnp.float32)
             + jnp.dot(xlo[0], wr[0:DH, :], preferred_element_type=jnp.float32)
             + jnp.dot(xhi[0], wr[DH:D, :], preferred_element_type=jnp.float32)
             + b[...])
        if final_softmax:
            m = jnp.max(h, axis=1, keepdims=True)
            e = jnp.exp(h - m)
            out[...] = h - (m + jnp.log(jnp.sum(e, axis=1, keepdims=True)))
        else:
            out[...] = jnp.where(h > 0, h, jnp.exp(h) - 1.0)

    half = pl.BlockSpec((1, blk, DH), lambda i: (0, i, 0))
    half2 = pl.BlockSpec((1, blk, DH), lambda i: (1, i, 0))
    return pl.pallas_call(
        body,
        grid=grid,
        in_specs=[half, half2,
                  pl.BlockSpec((blk, 8), lambda i: (i, 0)),
                  half, half2,
                  pl.BlockSpec((D, D), lambda i: (0, 0)),
                  pl.BlockSpec((D, D), lambda i: (0, 0)),
                  pl.BlockSpec((1, D), lambda i: (0, 0))],
        out_specs=pl.BlockSpec((blk, D), lambda i: (i, 0)),
        out_shape=jax.ShapeDtypeStruct((n_rows, D), jnp.float32),
    )


M1P = 20480
M2P = 10240
N1G = NW * CH * _cdiv(N1, NW * CH)
E1P = NS * CH * SB * _cdiv(E1, NS * CH * SB)
E2P = NS * CH * SB * _cdiv(E2, NS * CH * SB)

_gather1 = _make_row_gather(N1G)
_edge1 = _make_edge_kernel(E1P, M1P, M1P)
_edge2 = _make_edge_kernel(E2P, M2P, M2P)
_dense1 = _make_dense_kernel(M1P, final_softmax=False)
_dense2 = _make_dense_kernel(M2P, final_softmax=True)


def kernel(x, n_id, res_n_id1, edge_index1, res_n_id2, edge_index2,
           W1, W1_root, b1, W2, W2_root, b2):
    z64 = jnp.zeros((CH, DH), jnp.float32)
    o8 = jnp.stack([jnp.ones((CH, 8), jnp.float32),
                    jnp.zeros((CH, 8), jnp.float32)])

    nid2 = jnp.concatenate([n_id.astype(jnp.int32),
                            jnp.zeros((N1G - N1,), jnp.int32)]
                           ).reshape(NW, N1G // NW)
    xb = _gather1(x, nid2)

    xb2 = xb.reshape(2 * N1G, DH)
    src1 = _pad_reshape_idx(edge_index1[0], E1P, 0)
    dst1 = _pad_reshape_idx(edge_index1[1], E1P, M1)
    res1 = _pad_reshape_idx(res_n_id1, M1P, 0)
    agg1, deg1, xdst1 = _edge1(xb2, src1, dst1, res1, z64, o8)
    h1 = _dense1(agg1, agg1, deg1, xdst1, xdst1, W1, W1_root,
                 b1.reshape(1, D))

    h12 = h1.reshape(2 * M1P, DH)
    src2 = _pad_reshape_idx(edge_index2[0], E2P, 0)
    dst2 = _pad_reshape_idx(edge_index2[1], E2P, M2)
    res2 = _pad_reshape_idx(res_n_id2, M2P, 0)
    agg2, deg2, xdst2 = _edge2(h12, src2, dst2, res2, z64, o8)
    out = _dense2(agg2, agg2, deg2, xdst2, xdst2, W2, W2_root,
                  b2.reshape(1, D))
    return out[:M2]


# speedup vs baseline: 1.0294x; 1.0294x over previous
import jax
import jax.numpy as jnp
from jax import lax
from jax.experimental import pallas as pl
from jax.experimental.pallas import tpu as pltpu
from jax.experimental.pallas import tpu_sc as plsc

N_NODES = 100000
D = 128
DH = 64
N1 = 50000
M1 = 20000
E1 = 320000
M2 = 10000
E2 = 160000

NS = 16
NC = 2
NW = NC * NS
CH = 128
NB = 4
SB = 16


def _cdiv(a, b):
    return (a + b - 1) // b


def _wait(src, dst, sem):
    pltpu.make_async_copy(src, dst, sem).wait()


def _make_row_gather(n_rows_out, kt0, nc0, kt1, nc1):
    assert NS * (kt0 + kt1) == n_rows_out
    g0, g1c = kt0 // nc0, kt1 // nc1
    mesh = plsc.VectorSubcoreMesh(core_axis_name="c", subcore_axis_name="s")

    def body(table, idx1, out, idx_v, r0, r1, s0, s1, q0, q1):
        rows = [r0, r1]
        wsem = [s0, s1]
        gsem = [q0, q1]
        cid = lax.axis_index("c")
        sid = lax.axis_index("s")

        def run(kt, nc, gc, obase):
            pltpu.sync_copy(idx1.at[pl.ds(obase, kt)], idx_v.at[pl.ds(0, kt)])

            def owrite(q):
                u = q % 2
                _wait(table.at[idx_v.at[pl.ds(0, gc)]], rows[u].at[pl.ds(0, gc)],
                      gsem[u])
                pltpu.async_copy(rows[u].at[pl.ds(0, gc)],
                                 out.at[pl.ds(obase + q * gc, gc)], wsem[u])

            for q in range(nc):
                u = q % 2
                if q >= 2:
                    _wait(rows[u].at[pl.ds(0, gc)], out.at[pl.ds(obase, gc)],
                          wsem[u])
                pltpu.async_copy(table.at[idx_v.at[pl.ds(q * gc, gc)]],
                                 rows[u].at[pl.ds(0, gc)], gsem[u])
                if q >= 1:
                    owrite(q - 1)
            owrite(nc - 1)
            for q in range(max(nc - 2, 0), nc):
                _wait(rows[q % 2].at[pl.ds(0, gc)], out.at[pl.ds(obase, gc)],
                      wsem[q % 2])

        @pl.when(cid == 0)
        def _():
            run(kt0, nc0, g0, sid * kt0)

        @pl.when(cid == 1)
        def _():
            run(kt1, nc1, g1c, NS * kt0 + sid * kt1)

    gmax = max(g0, g1c)
    return pl.kernel(
        body,
        out_type=jax.ShapeDtypeStruct((n_rows_out, D), jnp.float32),
        mesh=mesh,
        compiler_params=pltpu.CompilerParams(use_tc_tiling_on_sc=False),
        scratch_types=(
            [pltpu.VMEM((max(kt0, kt1),), jnp.int32)]
            + [pltpu.VMEM((gmax, D), jnp.float32)] * 2
            + [pltpu.SemaphoreType.DMA] * 4
        ),
    )


def _make_edge_kernel(n_edges, n_dst_pad, n_res_pad):
    ke = n_edges // (NS * CH)
    nb = ke // SB
    assert ke % SB == 0 and SB % NB == 0
    kr = n_res_pad // (NS * CH)
    stripe = n_dst_pad // NS
    ks = stripe // CH
    assert kr == ks and kr >= NB and kr <= SB

    mesh = plsc.VectorSubcoreMesh(core_axis_name="c", subcore_axis_name="s")

    out_type = (
        jax.ShapeDtypeStruct((NC, n_dst_pad, DH), jnp.float32),
        jax.ShapeDtypeStruct((n_dst_pad, 8), jnp.float32),
        jax.ShapeDtypeStruct((NC, n_res_pad, DH), jnp.float32),
    )

    def body(table, src3, dst3, res3, z64, o8,
             agg_out, deg_out, xdst_out,
             src_b, dst_b, r0, r1, r2, r3, ones_v,
             agg_sh, deg_sh, s0, s1, s2, s3, g0, g1, g2, g3):
        rows = [r0, r1, r2, r3]
        ssem = [s0, s1, s2, s3]
        gsem = [g0, g1, g2, g3]
        cid = lax.axis_index("c")
        sid = lax.axis_index("s")
        base = sid * stripe

        pltpu.sync_copy(z64, rows[0])
        for r in range(ks):
            pltpu.sync_copy(rows[0], agg_sh.at[pl.ds(base + r * CH, CH)])

        @pl.when(cid == 0)
        def _():
            for r in range(ks):
                pltpu.sync_copy(o8.at[1], deg_sh.at[pl.ds(base + r * CH, CH)])
        pltpu.sync_copy(o8.at[0], ones_v)

        def compose(idx_v, nk):
            def jbody(j, _):
                for i in range(CH // 16):
                    s16 = idx_v[j, pl.ds(i * 16, 16)]
                    idx_v[j, pl.ds(i * 16, 16)] = s16 * 2 + cid
                return 0
            lax.fori_loop(0, nk, jbody, 0)

        plsc.subcore_barrier()

        def consume(jj, u):
            _wait(table.at[src_b.at[0]], rows[u], gsem[u])
            pltpu.async_copy(rows[u], agg_sh.at[dst_b.at[jj]], ssem[u],
                             add=True)

            @pl.when(cid == 0)
            def _():
                pltpu.sync_copy(ones_v, deg_sh.at[dst_b.at[jj]], add=True)

        def block_body(b, _):
            @pl.when(b >= 1)
            def _():
                consume(SB - 1, (SB - 1) % NB)

            pltpu.sync_copy(src3.at[sid, pl.ds(b * SB, SB)], src_b)
            pltpu.sync_copy(dst3.at[sid, pl.ds(b * SB, SB)], dst_b)
            compose(src_b, SB)

            def group_body(g, _):
                for u in range(NB):
                    j2 = g * NB + u
                    j = b * SB + j2

                    @pl.when(j >= NB)
                    def _():
                        _wait(rows[u], agg_sh.at[dst_b.at[0]], ssem[u])

                    pltpu.async_copy(table.at[src_b.at[j2]], rows[u], gsem[u])
                    if u > 0:
                        consume(j2 - 1, (u + NB - 1) % NB)
                    else:
                        @pl.when(g >= 1)
                        def _():
                            consume(j2 - 1, NB - 1)
                return 0
            lax.fori_loop(0, SB // NB, group_body, 0)
            return 0
        lax.fori_loop(0, nb, block_body, 0)

        consume(SB - 1, (SB - 1) % NB)
        for u in range(NB):
            _wait(rows[u], agg_sh.at[dst_b.at[0]], ssem[u])

        pltpu.sync_copy(res3.at[sid], src_b.at[pl.ds(0, kr)])
        compose(src_b, kr)

        def rwrite(j):
            u = j % NB
            _wait(table.at[src_b.at[0]], rows[u], gsem[u])
            pltpu.async_copy(rows[u],
                             xdst_out.at[cid, pl.ds(base + j * CH, CH)],
                             ssem[u])

        for j in range(kr):
            u = j % NB
            if j >= NB:
                _wait(rows[u], xdst_out.at[cid, pl.ds(base, CH)], ssem[u])
            pltpu.async_copy(table.at[src_b.at[j]], rows[u], gsem[u])
            if j >= 1:
                rwrite(j - 1)
        rwrite(kr - 1)
        for j in range(kr - NB, kr):
            u = j % NB
            _wait(rows[u], xdst_out.at[cid, pl.ds(base, CH)], ssem[u])

        plsc.subcore_barrier()

        pltpu.sync_copy(agg_sh.at[pl.ds(base, stripe)],
                        agg_out.at[cid, pl.ds(base, stripe)])

        @pl.when(cid == 0)
        def _():
            pltpu.sync_copy(deg_sh.at[pl.ds(base, stripe)],
                            deg_out.at[pl.ds(base, stripe)])

    return pl.kernel(
        body,
        out_type=out_type,
        mesh=mesh,
        compiler_params=pltpu.CompilerParams(use_tc_tiling_on_sc=False),
        scratch_types=(
            [pltpu.VMEM((SB, CH), jnp.int32),
             pltpu.VMEM((SB, CH), jnp.int32)]
            + [pltpu.VMEM((CH, DH), jnp.float32)] * NB
            + [pltpu.VMEM((CH, 8), jnp.float32),
               pltpu.VMEM_SHARED((n_dst_pad, DH), jnp.float32),
               pltpu.VMEM_SHARED((n_dst_pad, 8), jnp.float32)]
            + [pltpu.SemaphoreType.DMA] * (2 * NB)
        ),
    )


def _pad_reshape_idx(idx, total, fill, nmaj=NS):
    pad = total - idx.shape[0]
    idx = jnp.concatenate([idx.astype(jnp.int32),
                           jnp.full((pad,), fill, jnp.int32)])
    return idx.reshape(nmaj, total // (nmaj * CH), CH)


def _make_dense_kernel(n_rows, final_softmax):
    blk = 512
    grid = (n_rows // blk,)

    def body(alo, ahi, deg, xlo, xhi, w, wr, b, out):
        scale = 1.0 / jnp.maximum(deg[:, 0:1], 1.0)
        h = (jnp.dot(alo[0] * scale, w[0:DH, :], preferred_element_type=jnp.float32)
             + jnp.dot(ahi[0] * scale, w[DH:D, :], preferred_element_type=jnp.float32)
             + jnp.dot(xlo[0], wr[0:DH, :], preferred_element_type=jnp.float32)
             + jnp.dot(xhi[0], wr[DH:D, :], preferred_element_type=jnp.float32)
             + b[...])
        if final_softmax:
            m = jnp.max(h, axis=1, keepdims=True)
            e = jnp.exp(h - m)
            out[...] = h - (m + jnp.log(jnp.sum(e, axis=1, keepdims=True)))
        else:
            out[...] = jnp.where(h > 0, h, jnp.exp(h) - 1.0)

    half = pl.BlockSpec((1, blk, DH), lambda i: (0, i, 0))
    half2 = pl.BlockSpec((1, blk, DH), lambda i: (1, i, 0))
    return pl.pallas_call(
        body,
        grid=grid,
        in_specs=[half, half2,
                  pl.BlockSpec((blk, 8), lambda i: (i, 0)),
                  half, half2,
                  pl.BlockSpec((D, D), lambda i: (0, 0)),
                  pl.BlockSpec((D, D), lambda i: (0, 0)),
                  pl.BlockSpec((1, D), lambda i: (0, 0))],
        out_specs=pl.BlockSpec((blk, D), lambda i: (i, 0)),
        out_shape=jax.ShapeDtypeStruct((n_rows, D), jnp.float32),
    )


M1P = 20480
M2P = 10240
N1G = NW * CH * _cdiv(N1, NW * CH)
E1P = NS * CH * SB * _cdiv(E1, NS * CH * SB)
E2P = NS * CH * SB * _cdiv(E2, NS * CH * SB)

_gather1 = _make_row_gather(N1G, 416, 1, 2912, 7)
_edge1 = _make_edge_kernel(E1P, M1P, M1P)
_edge2 = _make_edge_kernel(E2P, M2P, M2P)
_dense1 = _make_dense_kernel(M1P, final_softmax=False)
_dense2 = _make_dense_kernel(M2P, final_softmax=True)


def kernel(x, n_id, res_n_id1, edge_index1, res_n_id2, edge_index2,
           W1, W1_root, b1, W2, W2_root, b2):
    z64 = jnp.zeros((CH, DH), jnp.float32)
    o8 = jnp.stack([jnp.ones((CH, 8), jnp.float32),
                    jnp.zeros((CH, 8), jnp.float32)])

    nid1 = jnp.concatenate([n_id.astype(jnp.int32),
                            jnp.zeros((N1G - N1,), jnp.int32)])
    xb = _gather1(x, nid1)

    xb2 = xb.reshape(2 * N1G, DH)
    src1 = _pad_reshape_idx(edge_index1[0], E1P, 0)
    dst1 = _pad_reshape_idx(edge_index1[1], E1P, M1)
    res1 = _pad_reshape_idx(res_n_id1, M1P, 0)
    agg1, deg1, xdst1 = _edge1(xb2, src1, dst1, res1, z64, o8)
    h1 = _dense1(agg1, agg1, deg1, xdst1, xdst1, W1, W1_root,
                 b1.reshape(1, D))

    h12 = h1.reshape(2 * M1P, DH)
    src2 = _pad_reshape_idx(edge_index2[0], E2P, 0)
    dst2 = _pad_reshape_idx(edge_index2[1], E2P, M2)
    res2 = _pad_reshape_idx(res_n_id2, M2P, 0)
    agg2, deg2, xdst2 = _edge2(h12, src2, dst2, res2, z64, o8)
    out = _dense2(agg2, agg2, deg2, xdst2, xdst2, W2, W2_root,
                  b2.reshape(1, D))
    return out[:M2]
